# baseline (speedup 1.0000x reference)
import functools

import jax
import jax.numpy as jnp
from jax import lax
from jax.experimental import pallas as pl
from jax.experimental.pallas import tpu as pltpu
from jax.experimental.pallas import tpu_sc as plsc

C, H, W = 9, 180, 240
NUM_B = 4
N_EVENTS = 262144

NROW = H * W
NROWP = 45056
VP = 16
ROWS_P = 2 * NROWP
TRASH = ROWS_P
SH_ROWS = ROWS_P + 8
STRIPE = ROWS_P // 16
CHUNK_ROWS = 512
N_CHUNK = STRIPE // CHUNK_ROWS

SUB = 128
N_SUBBATCH = N_EVENTS // SUB
SUBB_PER_TILE = N_SUBBATCH // 16
KB = 4
GROUPS = SUBB_PER_TILE // KB
NPASS = 2

EV_CHUNK = 2048
R2 = EV_CHUNK // 128


def _stats_body(t_ref, b_ref, out_ref):
    t = t_ref[...]
    b = b_ref[...]
    rows = []
    for k in range(NUM_B):
        mk = jnp.max(jnp.where(b == float(k), t, 0.0))
        rows.append(jnp.full((1, 128), mk, dtype=jnp.float32))
    rows.append(jnp.full((1, 128), jnp.min(b), dtype=jnp.float32))
    rows.append(jnp.zeros((3, 128), dtype=jnp.float32))
    out_ref[...] = jnp.concatenate(rows, axis=0)


def _stats(t2, b2):
    return pl.pallas_call(
        _stats_body,
        out_shape=jax.ShapeDtypeStruct((8, 128), jnp.float32),
    )(t2, b2)


def _values_body(stats_ref, w1r_ref, b1r_ref, w2aug_ref, b2r_ref, w3c_ref,
                 b3_ref, t_ref, b_ref, xi_ref, yi_ref, pi_ref, bi_ref,
                 vals_ref, i00_ref, i01_ref, i10_ref, i11_ref):
    bcol = b_ref[...]
    t = t_ref[...]
    tm = stats_ref[0, 0]
    for k in range(1, NUM_B):
        tm = jnp.where(bcol == k, stats_ref[k, 0], tm)
    tn = t / tm

    w1r = w1r_ref[...]
    b1r = b1r_ref[...]
    w2t = w2aug_ref[...]
    b2r = b2r_ref[...]
    w3c = w3c_ref[...]
    b3 = b3_ref[0, 0]
    a = tn * w1r
    h1s = []
    for i in range(C):
        z = a + (b1r - (i / (C - 1)) * w1r)
        h1s.append(jnp.maximum(z, 0.1 * z))
    h1 = jnp.concatenate(h1s, axis=0)
    h2p = jnp.dot(h1, w2t, preferred_element_type=jnp.float32) + b2r
    h2 = jnp.maximum(h2p, 0.1 * h2p)
    v = jnp.dot(h2, w3c, preferred_element_type=jnp.float32) + b3
    cols = [tn * v[i * EV_CHUNK:(i + 1) * EV_CHUNK] for i in range(C)]
    cols.append(jnp.zeros((EV_CHUNK, VP - C), dtype=jnp.float32))
    vals_ref[...] = jnp.concatenate(cols, axis=1)

    bmin = stats_ref[NUM_B, 0].astype(jnp.int32)
    combo = (bi_ref[...] - bmin) * 2 + pi_ref[...]
    dst = yi_ref[...] * W + xi_ref[...]
    for ref, base in ((i00_ref, 0), (i01_ref, 2), (i10_ref, 4), (i11_ref, 6)):
        owned = (combo >= base) & (combo < base + 2)
        ref[...] = jnp.where(owned, (combo - base) * NROWP + dst, TRASH)


def _values(stats, w1r, b1r, w2t, b2r, w3c, b3s, t_col, b_col,
            xi2, yi2, pi2, bi2):
    n_steps = N_EVENTS // EV_CHUNK
    small = lambda shp: pl.BlockSpec(shp, lambda j: (0, 0))
    col = pl.BlockSpec((EV_CHUNK, 1), lambda j: (j, 0))
    two_d = pl.BlockSpec((R2, 128), lambda j: (j, 0))
    idx_shape = jax.ShapeDtypeStruct((N_SUBBATCH, 128), jnp.int32)
    return pl.pallas_call(
        _values_body,
        grid=(n_steps,),
        in_specs=[
            small((8, 128)), small((1, 100)), small((1, 100)),
            small((100, 100)), small((1, 100)), small((100, 1)),
            small((1, 1)),
            col, col, two_d, two_d, two_d, two_d,
        ],
        out_specs=[
            pl.BlockSpec((EV_CHUNK, VP), lambda j: (j, 0)),
            two_d, two_d, two_d, two_d,
        ],
        out_shape=[
            jax.ShapeDtypeStruct((N_EVENTS, VP), jnp.float32),
            idx_shape, idx_shape, idx_shape, idx_shape,
        ],
    )(stats, w1r, b1r, w2t, b2r, w3c, b3s, t_col, b_col, xi2, yi2, pi2, bi2)


def _sc_body(vals_hbm, idx_hbm, zeros_hbm, out_hbm, shared, idx_v, vals_v,
             zv, cv, d0, d1, s0, s1):
    c = lax.axis_index("c")
    s = lax.axis_index("s")
    dsem = (d0, d1)
    ssem = (s0, s1)
    pltpu.sync_copy(zeros_hbm, zv)

    for p in range(NPASS):
        if p:
            plsc.subcore_barrier()
        for k in range(N_CHUNK):
            pltpu.sync_copy(zv, shared.at[pl.ds(s * STRIPE + k * CHUNK_ROWS,
                                                CHUNK_ROWS)])
        plsc.subcore_barrier()

        def start(g):
            buf = g & 1
            row0 = s * SUBB_PER_TILE + g * KB
            return (
                pltpu.async_copy(idx_hbm.at[2 * p + c, pl.ds(row0, KB)],
                                 idx_v.at[buf], dsem[buf]),
                pltpu.async_copy(vals_hbm.at[pl.ds(row0, KB)],
                                 vals_v.at[buf], dsem[buf]),
            )

        pend_dma = {0: start(0)}
        pend_sc = {}
        for g in range(GROUPS):
            buf = g & 1
            for dsc in pend_dma.pop(g):
                dsc.wait()
            if g - 1 in pend_sc:
                for dsc in pend_sc.pop(g - 1):
                    dsc.wait()
            if g + 1 < GROUPS:
                pend_dma[g + 1] = start(g + 1)
            pend_sc[g] = [
                pltpu.async_copy(vals_v.at[buf, j],
                                 shared.at[idx_v.at[buf, j]],
                                 ssem[buf], add=True)
                for j in range(KB)
            ]
        for g in list(pend_sc):
            for dsc in pend_sc.pop(g):
                dsc.wait()
        plsc.subcore_barrier()
        for k in range(N_CHUNK):
            base = s * STRIPE + k * CHUNK_ROWS
            pltpu.sync_copy(shared.at[pl.ds(base, CHUNK_ROWS)], cv)
            pltpu.sync_copy(cv, out_hbm.at[pl.ds((p * 2 + c) * ROWS_P + base,
                                                 CHUNK_ROWS)])


@functools.cache
def _get_sc_scatter():
    return functools.partial(
        pl.kernel,
        out_type=jax.ShapeDtypeStruct((NPASS * 2 * ROWS_P, VP), jnp.float32),
        mesh=plsc.VectorSubcoreMesh(core_axis_name="c", subcore_axis_name="s"),
        scratch_types=[
            pltpu.VMEM_SHARED((SH_ROWS, VP), jnp.float32),
            pltpu.VMEM((2, KB, SUB), jnp.int32),
            pltpu.VMEM((2, KB, SUB, VP), jnp.float32),
            pltpu.VMEM((CHUNK_ROWS, VP), jnp.float32),
            pltpu.VMEM((CHUNK_ROWS, VP), jnp.float32),
            pltpu.SemaphoreType.DMA,
            pltpu.SemaphoreType.DMA,
            pltpu.SemaphoreType.DMA,
            pltpu.SemaphoreType.DMA,
        ],
        compiler_params=pltpu.CompilerParams(use_tc_tiling_on_sc=False),
    )(_sc_body)


def kernel(events, W1, b1, W2, b2, W3, b3):
    ev = events.reshape(-1, 5)
    t = ev[:, 2]
    b_f = ev[:, 4]
    xi = ev[:, 0].astype(jnp.int32)
    yi = ev[:, 1].astype(jnp.int32)
    pi = ((ev[:, 3] + 1.0) * 0.5).astype(jnp.int32)
    bi = b_f.astype(jnp.int32)

    stats = _stats(t.reshape(2048, 128), b_f.reshape(2048, 128))

    vals, i00, i01, i10, i11 = _values(
        stats,
        W1.reshape(1, 100), b1.reshape(1, 100),
        W2.T, b2.reshape(1, 100),
        W3.reshape(100, 1), b3.reshape(1, 1),
        t.reshape(N_EVENTS, 1), bi.reshape(N_EVENTS, 1),
        xi.reshape(2048, 128), yi.reshape(2048, 128),
        pi.reshape(2048, 128), bi.reshape(2048, 128),
    )

    vals3 = vals.reshape(N_SUBBATCH, SUB, VP)
    idx_all = jnp.stack([i00, i10, i01, i11])
    zeros = jnp.zeros((CHUNK_ROWS, VP), dtype=jnp.float32)

    o = _get_sc_scatter()(vals3, idx_all, zeros)
    o = o.reshape(NPASS, 2, 2, NROWP, VP)

    planes = jnp.stack([o[0, 0, 0], o[0, 0, 1], o[1, 0, 0], o[1, 0, 1],
                        o[0, 1, 0], o[0, 1, 1], o[1, 1, 0], o[1, 1, 1]],
                       axis=0)
    vox = planes[:, :NROW, :C]
    vox = vox.reshape(NUM_B, 2, NROW, C).transpose(0, 1, 3, 2)
    return vox.reshape(NUM_B, 2 * C, H, W)


# speedup vs baseline: 1.0985x; 1.0985x over previous
import functools

import jax
import jax.numpy as jnp
from jax import lax
from jax.experimental import pallas as pl
from jax.experimental.pallas import tpu as pltpu
from jax.experimental.pallas import tpu_sc as plsc

C, H, W = 9, 180, 240
NUM_B = 4
N_EVENTS = 262144

NROW = H * W
NROWP = 45056
VP = 16
ROWS_P = 2 * NROWP
TRASH = ROWS_P
SH_ROWS = ROWS_P + 8
STRIPE = ROWS_P // 16
CHUNK_ROWS = 512
N_CHUNK = STRIPE // CHUNK_ROWS

SUB = 128
N_SUBBATCH = N_EVENTS // SUB
SUBB_PER_TILE = N_SUBBATCH // 16
KB = 4
GROUPS = SUBB_PER_TILE // KB
NPASS = 2

EV_CHUNK = 2048
R2 = EV_CHUNK // 128


def _stats_body(t_ref, b_ref, out_ref):
    t = t_ref[...]
    b = b_ref[...]
    rows = []
    for k in range(NUM_B):
        mk = jnp.max(jnp.where(b == float(k), t, 0.0))
        rows.append(jnp.full((1, 128), mk, dtype=jnp.float32))
    rows.append(jnp.full((1, 128), jnp.min(b), dtype=jnp.float32))
    rows.append(jnp.zeros((3, 128), dtype=jnp.float32))
    out_ref[...] = jnp.concatenate(rows, axis=0)


def _stats(t2, b2):
    return pl.pallas_call(
        _stats_body,
        out_shape=jax.ShapeDtypeStruct((8, 128), jnp.float32),
    )(t2, b2)


def _values_body(stats_ref, w1r_ref, b1r_ref, w2aug_ref, b2r_ref, w3c_ref,
                 b3_ref, t_ref, b_ref, xi_ref, yi_ref, pi_ref, bi_ref,
                 vals_ref, i00_ref, i01_ref, i10_ref, i11_ref):
    bcol = b_ref[...]
    t = t_ref[...]
    tm = stats_ref[0, 0]
    for k in range(1, NUM_B):
        tm = jnp.where(bcol == k, stats_ref[k, 0], tm)
    tn = t / tm

    w1r = w1r_ref[...]
    b1r = b1r_ref[...]
    w2t = w2aug_ref[...]
    b2r = b2r_ref[...]
    w3c = w3c_ref[...]
    b3 = b3_ref[0, 0]
    a = tn * w1r
    h1s = []
    for i in range(C):
        z = a + (b1r - (i / (C - 1)) * w1r)
        h1s.append(jnp.maximum(z, 0.1 * z))
    h1 = jnp.concatenate(h1s, axis=0)
    h2p = jnp.dot(h1, w2t, preferred_element_type=jnp.float32) + b2r
    h2 = jnp.maximum(h2p, 0.1 * h2p)
    v = jnp.dot(h2, w3c, preferred_element_type=jnp.float32) + b3
    cols = [tn * v[i * EV_CHUNK:(i + 1) * EV_CHUNK] for i in range(C)]
    cols.append(jnp.zeros((EV_CHUNK, VP - C), dtype=jnp.float32))
    vals_ref[...] = jnp.concatenate(cols, axis=1)

    bmin = stats_ref[NUM_B, 0].astype(jnp.int32)
    combo = (bi_ref[...] - bmin) * 2 + pi_ref[...]
    dst = yi_ref[...] * W + xi_ref[...]
    for ref, base in ((i00_ref, 0), (i01_ref, 2), (i10_ref, 4), (i11_ref, 6)):
        owned = (combo >= base) & (combo < base + 2)
        ref[...] = jnp.where(owned, (combo - base) * NROWP + dst, TRASH)


def _values(stats, w1r, b1r, w2t, b2r, w3c, b3s, t_col, b_col,
            xi2, yi2, pi2, bi2):
    n_steps = N_EVENTS // EV_CHUNK
    small = lambda shp: pl.BlockSpec(shp, lambda j: (0, 0))
    col = pl.BlockSpec((EV_CHUNK, 1), lambda j: (j, 0))
    two_d = pl.BlockSpec((R2, 128), lambda j: (j, 0))
    idx_shape = jax.ShapeDtypeStruct((N_SUBBATCH, 128), jnp.int32)
    return pl.pallas_call(
        _values_body,
        grid=(n_steps,),
        in_specs=[
            small((8, 128)), small((1, 100)), small((1, 100)),
            small((100, 100)), small((1, 100)), small((100, 1)),
            small((1, 1)),
            col, col, two_d, two_d, two_d, two_d,
        ],
        out_specs=[
            pl.BlockSpec((EV_CHUNK, VP), lambda j: (j, 0)),
            two_d, two_d, two_d, two_d,
        ],
        out_shape=[
            jax.ShapeDtypeStruct((N_EVENTS, VP), jnp.float32),
            idx_shape, idx_shape, idx_shape, idx_shape,
        ],
    )(stats, w1r, b1r, w2t, b2r, w3c, b3s, t_col, b_col, xi2, yi2, pi2, bi2)


def _sc_body(vals_hbm, idx_hbm, zeros_hbm, out_hbm, shared, idx_v, vals_v,
             zv, cv, d0, d1, s0, s1):
    c = lax.axis_index("c")
    s = lax.axis_index("s")
    dsem = (d0, d1)
    ssem = (s0, s1)
    pltpu.sync_copy(zeros_hbm, zv)

    for k in range(N_CHUNK):
        pltpu.sync_copy(zv, shared.at[pl.ds(s * STRIPE + k * CHUNK_ROWS,
                                            CHUNK_ROWS)])
    plsc.subcore_barrier()

    def start(g):
        buf = g & 1
        row0 = s * SUBB_PER_TILE + g * KB
        return (
            pltpu.async_copy(idx_hbm.at[c, pl.ds(row0, KB)],
                             idx_v.at[buf], dsem[buf]),
            pltpu.async_copy(vals_hbm.at[pl.ds(row0, KB)],
                             vals_v.at[buf], dsem[buf]),
        )

    pend_dma = {0: start(0)}
    pend_sc = {}
    for g in range(GROUPS):
        buf = g & 1
        for dsc in pend_dma.pop(g):
            dsc.wait()
        if g - 1 in pend_sc:
            for dsc in pend_sc.pop(g - 1):
                dsc.wait()
        if g + 1 < GROUPS:
            pend_dma[g + 1] = start(g + 1)
        pend_sc[g] = [
            pltpu.async_copy(vals_v.at[buf, j],
                             shared.at[idx_v.at[buf, j]],
                             ssem[buf], add=True)
            for j in range(KB)
        ]
    for g in list(pend_sc):
        for dsc in pend_sc.pop(g):
            dsc.wait()
    plsc.subcore_barrier()
    for k in range(N_CHUNK):
        base = s * STRIPE + k * CHUNK_ROWS
        pltpu.sync_copy(shared.at[pl.ds(base, CHUNK_ROWS)], cv)
        pltpu.sync_copy(cv, out_hbm.at[pl.ds(c * ROWS_P + base, CHUNK_ROWS)])


@functools.cache
def _get_sc_scatter():
    return functools.partial(
        pl.kernel,
        out_type=jax.ShapeDtypeStruct((2 * ROWS_P, VP), jnp.float32),
        mesh=plsc.VectorSubcoreMesh(core_axis_name="c", subcore_axis_name="s"),
        scratch_types=[
            pltpu.VMEM_SHARED((SH_ROWS, VP), jnp.float32),
            pltpu.VMEM((2, KB, SUB), jnp.int32),
            pltpu.VMEM((2, KB, SUB, VP), jnp.float32),
            pltpu.VMEM((CHUNK_ROWS, VP), jnp.float32),
            pltpu.VMEM((CHUNK_ROWS, VP), jnp.float32),
            pltpu.SemaphoreType.DMA,
            pltpu.SemaphoreType.DMA,
            pltpu.SemaphoreType.DMA,
            pltpu.SemaphoreType.DMA,
        ],
        compiler_params=pltpu.CompilerParams(use_tc_tiling_on_sc=False),
    )(_sc_body)


def kernel(events, W1, b1, W2, b2, W3, b3):
    ev = events.reshape(-1, 5)
    t = ev[:, 2]
    b_f = ev[:, 4]
    xi = ev[:, 0].astype(jnp.int32)
    yi = ev[:, 1].astype(jnp.int32)
    pi = ((ev[:, 3] + 1.0) * 0.5).astype(jnp.int32)
    bi = b_f.astype(jnp.int32)

    stats = _stats(t.reshape(2048, 128), b_f.reshape(2048, 128))

    vals, i00, i01, i10, i11 = _values(
        stats,
        W1.reshape(1, 100), b1.reshape(1, 100),
        W2.T, b2.reshape(1, 100),
        W3.reshape(100, 1), b3.reshape(1, 1),
        t.reshape(N_EVENTS, 1), bi.reshape(N_EVENTS, 1),
        xi.reshape(2048, 128), yi.reshape(2048, 128),
        pi.reshape(2048, 128), bi.reshape(2048, 128),
    )

    vals3 = vals.reshape(N_SUBBATCH, SUB, VP)
    idx_p0 = jnp.stack([i00, i10])
    idx_p1 = jnp.stack([i01, i11])
    zeros = jnp.zeros((CHUNK_ROWS, VP), dtype=jnp.float32)

    scat = _get_sc_scatter()
    o0 = scat(vals3, idx_p0, zeros).reshape(2, 2, NROWP, VP)
    o1 = scat(vals3, idx_p1, zeros).reshape(2, 2, NROWP, VP)

    planes = jnp.stack([o0[0, 0], o0[0, 1], o1[0, 0], o1[0, 1],
                        o0[1, 0], o0[1, 1], o1[1, 0], o1[1, 1]], axis=0)
    vox = planes[:, :NROW, :C]
    vox = vox.reshape(NUM_B, 2, NROW, C).transpose(0, 1, 3, 2)
    return vox.reshape(NUM_B, 2 * C, H, W)


# speedup vs baseline: 1.1599x; 1.0559x over previous
import functools

import jax
import jax.numpy as jnp
from jax import lax
from jax.experimental import pallas as pl
from jax.experimental.pallas import tpu as pltpu
from jax.experimental.pallas import tpu_sc as plsc

C, H, W = 9, 180, 240
NUM_B = 4
N_EVENTS = 262144

NROW = H * W
NROWP = 45056
VP = 16
ROWS_P = 2 * NROWP
TRASH = ROWS_P
SH_ROWS = ROWS_P + 8
STRIPE = ROWS_P // 16
CHUNK_ROWS = 512
N_CHUNK = STRIPE // CHUNK_ROWS

SUB = 128
N_SUBBATCH = N_EVENTS // SUB
SUBB_PER_TILE = N_SUBBATCH // 16
KB = 4
GROUPS = SUBB_PER_TILE // KB
NPASS = 2

EV_CHUNK = 2048
R2 = EV_CHUNK // 128


def _stats_body(t_ref, b_ref, out_ref):
    t = t_ref[...]
    b = b_ref[...]
    rows = []
    for k in range(NUM_B):
        mk = jnp.max(jnp.where(b == float(k), t, 0.0))
        rows.append(jnp.full((1, 128), mk, dtype=jnp.float32))
    rows.append(jnp.full((1, 128), jnp.min(b), dtype=jnp.float32))
    rows.append(jnp.zeros((3, 128), dtype=jnp.float32))
    out_ref[...] = jnp.concatenate(rows, axis=0)


def _stats(t2, b2):
    return pl.pallas_call(
        _stats_body,
        out_shape=jax.ShapeDtypeStruct((8, 128), jnp.float32),
    )(t2, b2)


def _values_body(stats_ref, w1r_ref, b1r_ref, w2aug_ref, b2r_ref, w3c_ref,
                 b3_ref, t_ref, b_ref, xi_ref, yi_ref, pi_ref, bi_ref,
                 vals_ref, i00_ref, i01_ref, i10_ref, i11_ref):
    bcol = b_ref[...]
    t = t_ref[...]
    tm = stats_ref[0, 0]
    for k in range(1, NUM_B):
        tm = jnp.where(bcol == float(k), stats_ref[k, 0], tm)
    tn = t / tm

    w1r = w1r_ref[...]
    b1r = b1r_ref[...]
    w2t = w2aug_ref[...]
    b2r = b2r_ref[...]
    w3c = w3c_ref[...]
    b3 = b3_ref[0, 0]
    a = tn * w1r
    h1s = []
    for i in range(C):
        z = a + (b1r - (i / (C - 1)) * w1r)
        h1s.append(jnp.maximum(z, 0.1 * z))
    h1 = jnp.concatenate(h1s, axis=0)
    h2p = jnp.dot(h1, w2t, preferred_element_type=jnp.float32) + b2r
    h2 = jnp.maximum(h2p, 0.1 * h2p)
    v = jnp.dot(h2, w3c, preferred_element_type=jnp.float32) + b3
    cols = [tn * v[i * EV_CHUNK:(i + 1) * EV_CHUNK] for i in range(C)]
    cols.append(jnp.zeros((EV_CHUNK, VP - C), dtype=jnp.float32))
    vals_ref[...] = jnp.concatenate(cols, axis=1)

    bmin = stats_ref[NUM_B, 0]
    combo = (bi_ref[...] - bmin) * 2.0 + (pi_ref[...] + 1.0) * 0.5
    dst = yi_ref[...] * float(W) + xi_ref[...]
    for ref, base in ((i00_ref, 0), (i01_ref, 2), (i10_ref, 4), (i11_ref, 6)):
        owned = (combo >= base) & (combo < base + 2)
        ref[...] = jnp.where(owned, (combo - base) * NROWP + dst,
                             float(TRASH)).astype(jnp.int32)


def _values(stats, w1r, b1r, w2t, b2r, w3c, b3s, t_col, b_col,
            xi2, yi2, pi2, bi2):
    n_steps = N_EVENTS // EV_CHUNK
    small = lambda shp: pl.BlockSpec(shp, lambda j: (0, 0))
    col = pl.BlockSpec((EV_CHUNK, 1), lambda j: (j, 0))
    two_d = pl.BlockSpec((R2, 128), lambda j: (j, 0))
    idx_shape = jax.ShapeDtypeStruct((N_SUBBATCH, 128), jnp.int32)
    return pl.pallas_call(
        _values_body,
        grid=(n_steps,),
        in_specs=[
            small((8, 128)), small((1, 100)), small((1, 100)),
            small((100, 100)), small((1, 100)), small((100, 1)),
            small((1, 1)),
            col, col, two_d, two_d, two_d, two_d,
        ],
        out_specs=[
            pl.BlockSpec((EV_CHUNK, VP), lambda j: (j, 0)),
            two_d, two_d, two_d, two_d,
        ],
        out_shape=[
            jax.ShapeDtypeStruct((N_EVENTS, VP), jnp.float32),
            idx_shape, idx_shape, idx_shape, idx_shape,
        ],
    )(stats, w1r, b1r, w2t, b2r, w3c, b3s, t_col, b_col, xi2, yi2, pi2, bi2)


def _sc_body(vals_hbm, idx_hbm, zeros_hbm, out_hbm, shared, idx_v, vals_v,
             zv, cv, d0, d1, s0, s1):
    c = lax.axis_index("c")
    s = lax.axis_index("s")
    dsem = (d0, d1)
    ssem = (s0, s1)
    pltpu.sync_copy(zeros_hbm, zv)

    for k in range(N_CHUNK):
        pltpu.sync_copy(zv, shared.at[pl.ds(s * STRIPE + k * CHUNK_ROWS,
                                            CHUNK_ROWS)])
    plsc.subcore_barrier()

    def start(g):
        buf = g & 1
        row0 = s * SUBB_PER_TILE + g * KB
        return (
            pltpu.async_copy(idx_hbm.at[c, pl.ds(row0, KB)],
                             idx_v.at[buf], dsem[buf]),
            pltpu.async_copy(vals_hbm.at[pl.ds(row0, KB)],
                             vals_v.at[buf], dsem[buf]),
        )

    pend_dma = {0: start(0)}
    pend_sc = {}
    for g in range(GROUPS):
        buf = g & 1
        for dsc in pend_dma.pop(g):
            dsc.wait()
        if g - 1 in pend_sc:
            for dsc in pend_sc.pop(g - 1):
                dsc.wait()
        if g + 1 < GROUPS:
            pend_dma[g + 1] = start(g + 1)
        pend_sc[g] = [
            pltpu.async_copy(vals_v.at[buf, j],
                             shared.at[idx_v.at[buf, j]],
                             ssem[buf], add=True)
            for j in range(KB)
        ]
    for g in list(pend_sc):
        for dsc in pend_sc.pop(g):
            dsc.wait()
    plsc.subcore_barrier()
    for k in range(N_CHUNK):
        base = s * STRIPE + k * CHUNK_ROWS
        pltpu.sync_copy(shared.at[pl.ds(base, CHUNK_ROWS)], cv)
        pltpu.sync_copy(cv, out_hbm.at[pl.ds(c * ROWS_P + base, CHUNK_ROWS)])


@functools.cache
def _get_sc_scatter():
    return functools.partial(
        pl.kernel,
        out_type=jax.ShapeDtypeStruct((2 * ROWS_P, VP), jnp.float32),
        mesh=plsc.VectorSubcoreMesh(core_axis_name="c", subcore_axis_name="s"),
        scratch_types=[
            pltpu.VMEM_SHARED((SH_ROWS, VP), jnp.float32),
            pltpu.VMEM((2, KB, SUB), jnp.int32),
            pltpu.VMEM((2, KB, SUB, VP), jnp.float32),
            pltpu.VMEM((CHUNK_ROWS, VP), jnp.float32),
            pltpu.VMEM((CHUNK_ROWS, VP), jnp.float32),
            pltpu.SemaphoreType.DMA,
            pltpu.SemaphoreType.DMA,
            pltpu.SemaphoreType.DMA,
            pltpu.SemaphoreType.DMA,
        ],
        compiler_params=pltpu.CompilerParams(use_tc_tiling_on_sc=False),
    )(_sc_body)


def kernel(events, W1, b1, W2, b2, W3, b3):
    ev = events.reshape(-1, 5)
    t = ev[:, 2]
    b_f = ev[:, 4]

    stats = _stats(t.reshape(2048, 128), b_f.reshape(2048, 128))

    vals, i00, i01, i10, i11 = _values(
        stats,
        W1.reshape(1, 100), b1.reshape(1, 100),
        W2.T, b2.reshape(1, 100),
        W3.reshape(100, 1), b3.reshape(1, 1),
        t.reshape(N_EVENTS, 1), b_f.reshape(N_EVENTS, 1),
        ev[:, 0].reshape(2048, 128), ev[:, 1].reshape(2048, 128),
        ev[:, 3].reshape(2048, 128), b_f.reshape(2048, 128),
    )

    vals3 = vals.reshape(N_SUBBATCH, SUB, VP)
    idx_p0 = jnp.stack([i00, i10])
    idx_p1 = jnp.stack([i01, i11])
    zeros = jnp.zeros((CHUNK_ROWS, VP), dtype=jnp.float32)

    scat = _get_sc_scatter()
    o0 = scat(vals3, idx_p0, zeros).reshape(2, 2, NROWP, VP)
    o1 = scat(vals3, idx_p1, zeros).reshape(2, 2, NROWP, VP)

    planes = jnp.stack([o0[0, 0], o0[0, 1], o1[0, 0], o1[0, 1],
                        o0[1, 0], o0[1, 1], o1[1, 0], o1[1, 1]], axis=0)
    vox = planes[:, :NROW, :C]
    vox = vox.reshape(NUM_B, 2, NROW, C).transpose(0, 1, 3, 2)
    return vox.reshape(NUM_B, 2 * C, H, W)
